# baseline (speedup 1.0000x reference)
import functools

import jax
import jax.numpy as jnp
from jax import lax
from jax.experimental import pallas as pl
from jax.experimental.pallas import tpu as pltpu
from jax.experimental.pallas import tpu_sc as plsc

N = 10000
E = 320000
F = 128

NC, NS = 2, 16
NW = NC * NS
BLK = 64
NBLK = 160
E_PAD = NW * NBLK * BLK
DEGW = 128
DB = 4
CH = 8
NCH = NBLK // CH
NPAD = NS * 632
RPT = NPAD // NS


@functools.cache
def _get_sc_segsum1():
    mesh = plsc.VectorSubcoreMesh(core_axis_name="c", subcore_axis_name="s",
                                  num_cores=NC, num_subcores=NS)
    return functools.partial(
        pl.kernel,
        out_type=jax.ShapeDtypeStruct((NC, NPAD, F), jnp.float32),
        mesh=mesh,
        scratch_types=[
            pltpu.VMEM((2, CH, BLK), jnp.int32),
            pltpu.VMEM((2, CH, BLK), jnp.int32),
            pltpu.VMEM((DB, BLK, F), jnp.float32),
            pltpu.VMEM_SHARED((NPAD, F), jnp.float32),
            [pltpu.SemaphoreType.DMA] * DB,
            [pltpu.SemaphoreType.DMA] * 2,
        ],
    )(_sc_segsum1_body)


def _sc_segsum1(table, src_b, dst_b, zer):
    return _get_sc_segsum1()(table, src_b, dst_b, zer)


def _sc_segsum1_body(table_hbm, srcb_hbm, dstb_hbm, zer_hbm, out_hbm,
                     sidx_v, didx_v, rows_v, acc_sh, gsem, isem):
    c = lax.axis_index("c")
    s = lax.axis_index("s")
    wid = c * NS + s
    pltpu.sync_copy(zer_hbm.at[pl.ds(s * RPT, RPT)],
                    acc_sh.at[pl.ds(s * RPT, RPT)])
    pltpu.sync_copy(srcb_hbm.at[wid].at[pl.ds(0, CH)], sidx_v.at[0])
    pltpu.sync_copy(dstb_hbm.at[wid].at[pl.ds(0, CH)], didx_v.at[0])
    plsc.subcore_barrier()

    def _idx_pair(kk, slot):
        return ((srcb_hbm.at[wid].at[pl.ds((kk + 1) * CH, CH)], sidx_v.at[slot]),
                (dstb_hbm.at[wid].at[pl.ds((kk + 1) * CH, CH)], didx_v.at[slot]))

    def _g(slot_ref, i, rs):
        return pltpu.async_copy(table_hbm.at[slot_ref.at[i]],
                                rows_v.at[rs], gsem[rs])

    for j in range(DB - 1):
        _g(sidx_v.at[0], j, j)

    @pl.loop(0, NCH, step=2)
    def _(k):
        for dk, slot in ((0, 0), (1, 1)):
            kk = k + dk
            nxt = 1 - slot
            @pl.when(kk + 1 < NCH)
            def _():
                for sr, ds in _idx_pair(kk, nxt):
                    pltpu.async_copy(sr, ds, isem[nxt])

            for i in range(CH):
                la = i + DB - 1
                if la < CH:
                    _g(sidx_v.at[slot], la, la % DB)
                else:
                    if la == CH:
                        @pl.when(kk + 1 < NCH)
                        def _():
                            for sr, ds in _idx_pair(kk, nxt):
                                pltpu.make_async_copy(sr, ds, isem[nxt]).wait()

                    @pl.when(kk + 1 < NCH)
                    def _():
                        _g(sidx_v.at[nxt], la - CH, la % DB)
                pltpu.make_async_copy(table_hbm.at[sidx_v.at[slot].at[i]],
                                      rows_v.at[i % DB], gsem[i % DB]).wait()
                pltpu.sync_copy(rows_v.at[i % DB],
                                acc_sh.at[didx_v.at[slot].at[i]], add=True)

    plsc.subcore_barrier()
    pltpu.sync_copy(acc_sh.at[pl.ds(s * RPT, RPT)],
                    out_hbm.at[c].at[pl.ds(s * RPT, RPT)])


@functools.cache
def _get_sc_degree():
    mesh = plsc.VectorSubcoreMesh(core_axis_name="c", subcore_axis_name="s",
                                  num_cores=NC, num_subcores=NS)
    return functools.partial(
        pl.kernel,
        out_type=jax.ShapeDtypeStruct((NC, NPAD, DEGW), jnp.float32),
        mesh=mesh,
        scratch_types=[
            pltpu.VMEM((NBLK, BLK), jnp.int32),
            pltpu.VMEM((BLK, DEGW), jnp.float32),
            pltpu.VMEM_SHARED((NPAD, DEGW), jnp.float32),
            pltpu.SemaphoreType.DMA,
        ],
    )(_sc_degree_body)


def _sc_degree(dst_b, ones, zer):
    return _get_sc_degree()(dst_b, ones, zer)


def _sc_degree_body(dstb_hbm, ones_hbm, zer_hbm, out_hbm, didx_v, ones_v, acc_sh, sem):
    c = lax.axis_index("c")
    s = lax.axis_index("s")
    wid = c * NS + s
    pltpu.sync_copy(zer_hbm.at[pl.ds(s * RPT, RPT)],
                    acc_sh.at[pl.ds(s * RPT, RPT)])
    pltpu.sync_copy(ones_hbm, ones_v)
    pltpu.sync_copy(dstb_hbm.at[wid], didx_v)
    plsc.subcore_barrier()

    @pl.loop(0, NBLK)
    def _(b):
        pltpu.sync_copy(ones_v, acc_sh.at[didx_v.at[b]], add=True)

    plsc.subcore_barrier()
    pltpu.sync_copy(acc_sh.at[pl.ds(s * RPT, RPT)],
                    out_hbm.at[c].at[pl.ds(s * RPT, RPT)])


_R = 10
_BR = N // _R


def _norm_of(degp_ref):
    deg = degp_ref[0, :, 0:1] + degp_ref[1, :, 0:1]
    return lax.rsqrt(jnp.maximum(deg, 1.0))


def _tc1a_body(x_ref, h_ref, wx0_ref, wh0_ref, bcat_ref, xw0_ref, hw0_ref):
    xw0_ref[...] = jnp.dot(x_ref[...], wx0_ref[...],
                           preferred_element_type=jnp.float32) + bcat_ref[...]
    hw0_ref[...] = jnp.dot(h_ref[...], wh0_ref[...],
                           preferred_element_type=jnp.float32)


def _tc1b_body(x_ref, h_ref, degp_ref, xn_ref, hn_ref):
    norm = _norm_of(degp_ref)
    xn_ref[...] = x_ref[...] * norm
    hn_ref[...] = h_ref[...] * norm


def _tc2_body(px_ref, ph_ref, degp_ref, xw0_ref, hw0_ref, h_ref,
              wx1_ref, wh1_ref, whh0_ref, z_ref, hrn_ref, t1_ref):
    norm = _norm_of(degp_ref)
    sx = (px_ref[0] + px_ref[1]) * norm
    sh = (ph_ref[0] + ph_ref[1]) * norm
    sxw = jnp.dot(sx, wx1_ref[...], preferred_element_type=jnp.float32)
    shw = jnp.dot(sh, wh1_ref[...], preferred_element_type=jnp.float32)
    xw0 = xw0_ref[...]
    hw0 = hw0_ref[...]
    z = jax.nn.sigmoid(xw0[:, 0:F] - sxw[:, 0:F] + hw0[:, 0:F] - shw[:, 0:F])
    r = jax.nn.sigmoid(xw0[:, F:2 * F] - sxw[:, F:2 * F]
                       + hw0[:, F:2 * F] - shw[:, F:2 * F])
    hr = h_ref[...] * r
    hrw = jnp.dot(hr, whh0_ref[...], preferred_element_type=jnp.float32)
    z_ref[...] = z
    hrn_ref[...] = hr * norm
    t1_ref[...] = xw0[:, 2 * F:3 * F] - sxw[:, 2 * F:3 * F] + hrw


def _tc3_body(phr_ref, degp_ref, t1_ref, z_ref, h_ref, whh1_ref, out_ref):
    norm = _norm_of(degp_ref)
    shr = (phr_ref[0] + phr_ref[1]) * norm
    ht = jnp.tanh(t1_ref[...] - jnp.dot(shr, whh1_ref[...],
                                        preferred_element_type=jnp.float32))
    z = z_ref[...]
    out_ref[...] = z * h_ref[...] + (1.0 - z) * ht


def _rows(i):
    return (i, 0)


def _full(i):
    return (0, 0)


def _part3(i):
    return (0, i, 0)


_row_spec = pl.BlockSpec((_BR, F), _rows)
_degp_spec = pl.BlockSpec((NC, _BR, DEGW), _part3)
_part_spec = pl.BlockSpec((NC, _BR, F), _part3)


def _tc1a(X, H, WX0, WH0, bcat):
    return pl.pallas_call(
        _tc1a_body,
        grid=(_R,),
        in_specs=[_row_spec, _row_spec,
                  pl.BlockSpec((F, 3 * F), _full),
                  pl.BlockSpec((F, 2 * F), _full),
                  pl.BlockSpec((1, 3 * F), _full)],
        out_specs=[pl.BlockSpec((_BR, 3 * F), _rows),
                   pl.BlockSpec((_BR, 2 * F), _rows)],
        out_shape=[jax.ShapeDtypeStruct((N, 3 * F), jnp.float32),
                   jax.ShapeDtypeStruct((N, 2 * F), jnp.float32)],
    )(X, H, WX0, WH0, bcat)


def _tc1b(X, H, degp):
    return pl.pallas_call(
        _tc1b_body,
        grid=(_R,),
        in_specs=[_row_spec, _row_spec, _degp_spec],
        out_specs=[_row_spec, _row_spec],
        out_shape=[jax.ShapeDtypeStruct((N, F), jnp.float32),
                   jax.ShapeDtypeStruct((N, F), jnp.float32)],
    )(X, H, degp)


def _tc2(px, ph, degp, xw0, hw0, H, WX1, WH1, Whh0):
    return pl.pallas_call(
        _tc2_body,
        grid=(_R,),
        in_specs=[_part_spec, _part_spec, _degp_spec,
                  pl.BlockSpec((_BR, 3 * F), _rows),
                  pl.BlockSpec((_BR, 2 * F), _rows),
                  _row_spec,
                  pl.BlockSpec((F, 3 * F), _full),
                  pl.BlockSpec((F, 2 * F), _full),
                  pl.BlockSpec((F, F), _full)],
        out_specs=[_row_spec, _row_spec, _row_spec],
        out_shape=[jax.ShapeDtypeStruct((N, F), jnp.float32)] * 3,
    )(px, ph, degp, xw0, hw0, H, WX1, WH1, Whh0)


def _tc3(phr, degp, t1, z, H, Whh1):
    return pl.pallas_call(
        _tc3_body,
        grid=(_R,),
        in_specs=[_part_spec, _degp_spec, _row_spec, _row_spec, _row_spec,
                  pl.BlockSpec((F, F), _full)],
        out_specs=_row_spec,
        out_shape=jax.ShapeDtypeStruct((N, F), jnp.float32),
    )(phr, degp, t1, z, H, Whh1)


def kernel(X, edge_index, H, Wxz, bxz, Whz, bhz, Wxr, bxr, Whr, bhr,
           Wxh, bxh, Whh, bhh):
    src = edge_index[0]
    dst = edge_index[1]
    pad = E_PAD - E
    pad_i = jnp.arange(pad, dtype=jnp.int32)
    src_b = jnp.concatenate(
        [src, pad_i % N]).reshape(NW, NBLK, BLK)
    dst_b = jnp.concatenate(
        [dst, N + pad_i % (NPAD - N)]).reshape(NW, NBLK, BLK)

    zer_rows = jnp.zeros((NPAD, F), jnp.float32)
    zer_deg = jnp.zeros((NPAD, DEGW), jnp.float32)
    ones_blk = jnp.ones((BLK, DEGW), jnp.float32)

    WX0 = jnp.concatenate([Wxz[:F], Wxr[:F], Wxh[:F]], axis=1)
    WX1 = jnp.concatenate([Wxz[F:], Wxr[F:], Wxh[F:]], axis=1)
    WH0 = jnp.concatenate([Whz[:F], Whr[:F]], axis=1)
    WH1 = jnp.concatenate([Whz[F:], Whr[F:]], axis=1)
    bcat = jnp.concatenate([bxz + bhz, bxr + bhr, bxh + bhh]).reshape(1, 3 * F)

    degp = _sc_degree(dst_b, ones_blk, zer_deg)
    xw0, hw0 = _tc1a(X, H, WX0, WH0, bcat)
    xn, hn = _tc1b(X, H, degp)
    px = _sc_segsum1(xn, src_b, dst_b, zer_rows)
    ph = _sc_segsum1(hn, src_b, dst_b, zer_rows)
    z, hrn, t1 = _tc2(px, ph, degp, xw0, hw0, H, WX1, WH1, Whh[:F])
    phr = _sc_segsum1(hrn, src_b, dst_b, zer_rows)
    return _tc3(phr, degp, t1, z, H, Whh[F:])


# speedup vs baseline: 1.0159x; 1.0159x over previous
import functools

import jax
import jax.numpy as jnp
from jax import lax
from jax.experimental import pallas as pl
from jax.experimental.pallas import tpu as pltpu
from jax.experimental.pallas import tpu_sc as plsc

N = 10000
E = 320000
F = 128

NC, NS = 2, 16
NW = NC * NS
BLK = 64
NBLK = 160
E_PAD = NW * NBLK * BLK
DEGW = 128
DB = 4
CH = 8
NCH = NBLK // CH
NPAD = NS * 632
RPT = NPAD // NS


@functools.cache
def _get_sc_segsum1():
    mesh = plsc.VectorSubcoreMesh(core_axis_name="c", subcore_axis_name="s",
                                  num_cores=NC, num_subcores=NS)
    return functools.partial(
        pl.kernel,
        out_type=jax.ShapeDtypeStruct((NC, NPAD, F), jnp.float32),
        mesh=mesh,
        scratch_types=[
            pltpu.VMEM((2, CH, BLK), jnp.int32),
            pltpu.VMEM((2, CH, BLK), jnp.int32),
            pltpu.VMEM((DB, BLK, F), jnp.float32),
            pltpu.VMEM_SHARED((NPAD, F), jnp.float32),
            [pltpu.SemaphoreType.DMA] * DB,
            [pltpu.SemaphoreType.DMA] * 2,
        ],
    )(_sc_segsum1_body)


def _sc_segsum1(table, src_b, dst_b):
    return _get_sc_segsum1()(table, src_b, dst_b)


def _sc_segsum1_body(table_hbm, srcb_hbm, dstb_hbm, out_hbm,
                     sidx_v, didx_v, rows_v, acc_sh, gsem, isem):
    c = lax.axis_index("c")
    s = lax.axis_index("s")
    wid = c * NS + s
    @pl.loop(0, BLK)
    def _(i):
        for j in range(F // 16):
            rows_v[0, i, pl.ds(j * 16, 16)] = jnp.zeros((16,), jnp.float32)
    for m in range(RPT // BLK):
        pltpu.sync_copy(rows_v.at[0],
                        acc_sh.at[pl.ds(s * RPT + m * BLK, BLK)])
    if RPT % BLK:
        pltpu.sync_copy(rows_v.at[0].at[pl.ds(0, RPT % BLK)],
                        acc_sh.at[pl.ds(s * RPT + (RPT // BLK) * BLK, RPT % BLK)])
    pltpu.sync_copy(srcb_hbm.at[wid].at[pl.ds(0, CH)], sidx_v.at[0])
    pltpu.sync_copy(dstb_hbm.at[wid].at[pl.ds(0, CH)], didx_v.at[0])
    plsc.subcore_barrier()

    def _idx_pair(kk, slot):
        return ((srcb_hbm.at[wid].at[pl.ds((kk + 1) * CH, CH)], sidx_v.at[slot]),
                (dstb_hbm.at[wid].at[pl.ds((kk + 1) * CH, CH)], didx_v.at[slot]))

    def _g(slot_ref, i, rs):
        return pltpu.async_copy(table_hbm.at[slot_ref.at[i]],
                                rows_v.at[rs], gsem[rs])

    for j in range(DB - 1):
        _g(sidx_v.at[0], j, j)

    @pl.loop(0, NCH, step=2)
    def _(k):
        for dk, slot in ((0, 0), (1, 1)):
            kk = k + dk
            nxt = 1 - slot
            @pl.when(kk + 1 < NCH)
            def _():
                for sr, ds in _idx_pair(kk, nxt):
                    pltpu.async_copy(sr, ds, isem[nxt])

            for i in range(CH):
                la = i + DB - 1
                if la < CH:
                    _g(sidx_v.at[slot], la, la % DB)
                else:
                    if la == CH:
                        @pl.when(kk + 1 < NCH)
                        def _():
                            for sr, ds in _idx_pair(kk, nxt):
                                pltpu.make_async_copy(sr, ds, isem[nxt]).wait()

                    @pl.when(kk + 1 < NCH)
                    def _():
                        _g(sidx_v.at[nxt], la - CH, la % DB)
                pltpu.make_async_copy(table_hbm.at[sidx_v.at[slot].at[i]],
                                      rows_v.at[i % DB], gsem[i % DB]).wait()
                pltpu.sync_copy(rows_v.at[i % DB],
                                acc_sh.at[didx_v.at[slot].at[i]], add=True)

    plsc.subcore_barrier()
    pltpu.sync_copy(acc_sh.at[pl.ds(s * RPT, RPT)],
                    out_hbm.at[c].at[pl.ds(s * RPT, RPT)])


@functools.cache
def _get_sc_degree():
    mesh = plsc.VectorSubcoreMesh(core_axis_name="c", subcore_axis_name="s",
                                  num_cores=NC, num_subcores=NS)
    return functools.partial(
        pl.kernel,
        out_type=jax.ShapeDtypeStruct((NC, NPAD, DEGW), jnp.float32),
        mesh=mesh,
        scratch_types=[
            pltpu.VMEM((NBLK, BLK), jnp.int32),
            pltpu.VMEM((BLK, DEGW), jnp.float32),
            pltpu.VMEM_SHARED((NPAD, DEGW), jnp.float32),
            pltpu.SemaphoreType.DMA,
        ],
    )(_sc_degree_body)


def _sc_degree(dst_b, ones, zer):
    return _get_sc_degree()(dst_b, ones, zer)


def _sc_degree_body(dstb_hbm, ones_hbm, zer_hbm, out_hbm, didx_v, ones_v, acc_sh, sem):
    c = lax.axis_index("c")
    s = lax.axis_index("s")
    wid = c * NS + s
    pltpu.sync_copy(zer_hbm.at[pl.ds(s * RPT, RPT)],
                    acc_sh.at[pl.ds(s * RPT, RPT)])
    pltpu.sync_copy(ones_hbm, ones_v)
    pltpu.sync_copy(dstb_hbm.at[wid], didx_v)
    plsc.subcore_barrier()

    @pl.loop(0, NBLK)
    def _(b):
        pltpu.sync_copy(ones_v, acc_sh.at[didx_v.at[b]], add=True)

    plsc.subcore_barrier()
    pltpu.sync_copy(acc_sh.at[pl.ds(s * RPT, RPT)],
                    out_hbm.at[c].at[pl.ds(s * RPT, RPT)])


_R = 10
_BR = N // _R


def _norm_of(degp_ref):
    deg = degp_ref[0, :, 0:1] + degp_ref[1, :, 0:1]
    return lax.rsqrt(jnp.maximum(deg, 1.0))


def _tc1a_body(x_ref, h_ref, wx0_ref, wh0_ref, bcat_ref, xw0_ref, hw0_ref):
    xw0_ref[...] = jnp.dot(x_ref[...], wx0_ref[...],
                           preferred_element_type=jnp.float32) + bcat_ref[...]
    hw0_ref[...] = jnp.dot(h_ref[...], wh0_ref[...],
                           preferred_element_type=jnp.float32)


def _tc1b_body(x_ref, h_ref, degp_ref, xn_ref, hn_ref):
    norm = _norm_of(degp_ref)
    xn_ref[...] = x_ref[...] * norm
    hn_ref[...] = h_ref[...] * norm


def _tc2_body(px_ref, ph_ref, degp_ref, xw0_ref, hw0_ref, h_ref,
              wx1_ref, wh1_ref, whh0_ref, z_ref, hrn_ref, t1_ref):
    norm = _norm_of(degp_ref)
    sx = (px_ref[0] + px_ref[1]) * norm
    sh = (ph_ref[0] + ph_ref[1]) * norm
    sxw = jnp.dot(sx, wx1_ref[...], preferred_element_type=jnp.float32)
    shw = jnp.dot(sh, wh1_ref[...], preferred_element_type=jnp.float32)
    xw0 = xw0_ref[...]
    hw0 = hw0_ref[...]
    z = jax.nn.sigmoid(xw0[:, 0:F] - sxw[:, 0:F] + hw0[:, 0:F] - shw[:, 0:F])
    r = jax.nn.sigmoid(xw0[:, F:2 * F] - sxw[:, F:2 * F]
                       + hw0[:, F:2 * F] - shw[:, F:2 * F])
    hr = h_ref[...] * r
    hrw = jnp.dot(hr, whh0_ref[...], preferred_element_type=jnp.float32)
    z_ref[...] = z
    hrn_ref[...] = hr * norm
    t1_ref[...] = xw0[:, 2 * F:3 * F] - sxw[:, 2 * F:3 * F] + hrw


def _tc3_body(phr_ref, degp_ref, t1_ref, z_ref, h_ref, whh1_ref, out_ref):
    norm = _norm_of(degp_ref)
    shr = (phr_ref[0] + phr_ref[1]) * norm
    ht = jnp.tanh(t1_ref[...] - jnp.dot(shr, whh1_ref[...],
                                        preferred_element_type=jnp.float32))
    z = z_ref[...]
    out_ref[...] = z * h_ref[...] + (1.0 - z) * ht


def _rows(i):
    return (i, 0)


def _full(i):
    return (0, 0)


def _part3(i):
    return (0, i, 0)


_row_spec = pl.BlockSpec((_BR, F), _rows)
_degp_spec = pl.BlockSpec((NC, _BR, DEGW), _part3)
_part_spec = pl.BlockSpec((NC, _BR, F), _part3)


def _tc1a(X, H, WX0, WH0, bcat):
    return pl.pallas_call(
        _tc1a_body,
        grid=(_R,),
        in_specs=[_row_spec, _row_spec,
                  pl.BlockSpec((F, 3 * F), _full),
                  pl.BlockSpec((F, 2 * F), _full),
                  pl.BlockSpec((1, 3 * F), _full)],
        out_specs=[pl.BlockSpec((_BR, 3 * F), _rows),
                   pl.BlockSpec((_BR, 2 * F), _rows)],
        out_shape=[jax.ShapeDtypeStruct((N, 3 * F), jnp.float32),
                   jax.ShapeDtypeStruct((N, 2 * F), jnp.float32)],
    )(X, H, WX0, WH0, bcat)


def _tc1b(X, H, degp):
    return pl.pallas_call(
        _tc1b_body,
        grid=(_R,),
        in_specs=[_row_spec, _row_spec, _degp_spec],
        out_specs=[_row_spec, _row_spec],
        out_shape=[jax.ShapeDtypeStruct((N, F), jnp.float32),
                   jax.ShapeDtypeStruct((N, F), jnp.float32)],
    )(X, H, degp)


def _tc2(px, ph, degp, xw0, hw0, H, WX1, WH1, Whh0):
    return pl.pallas_call(
        _tc2_body,
        grid=(_R,),
        in_specs=[_part_spec, _part_spec, _degp_spec,
                  pl.BlockSpec((_BR, 3 * F), _rows),
                  pl.BlockSpec((_BR, 2 * F), _rows),
                  _row_spec,
                  pl.BlockSpec((F, 3 * F), _full),
                  pl.BlockSpec((F, 2 * F), _full),
                  pl.BlockSpec((F, F), _full)],
        out_specs=[_row_spec, _row_spec, _row_spec],
        out_shape=[jax.ShapeDtypeStruct((N, F), jnp.float32)] * 3,
    )(px, ph, degp, xw0, hw0, H, WX1, WH1, Whh0)


def _tc3(phr, degp, t1, z, H, Whh1):
    return pl.pallas_call(
        _tc3_body,
        grid=(_R,),
        in_specs=[_part_spec, _degp_spec, _row_spec, _row_spec, _row_spec,
                  pl.BlockSpec((F, F), _full)],
        out_specs=_row_spec,
        out_shape=jax.ShapeDtypeStruct((N, F), jnp.float32),
    )(phr, degp, t1, z, H, Whh1)


def kernel(X, edge_index, H, Wxz, bxz, Whz, bhz, Wxr, bxr, Whr, bhr,
           Wxh, bxh, Whh, bhh):
    src = edge_index[0]
    dst = edge_index[1]
    pad = E_PAD - E
    pad_i = jnp.arange(pad, dtype=jnp.int32)
    src_b = jnp.concatenate(
        [src, pad_i % N]).reshape(NW, NBLK, BLK)
    dst_b = jnp.concatenate(
        [dst, N + pad_i % (NPAD - N)]).reshape(NW, NBLK, BLK)

    zer_deg = jnp.zeros((NPAD, DEGW), jnp.float32)
    ones_blk = jnp.ones((BLK, DEGW), jnp.float32)

    WX0 = jnp.concatenate([Wxz[:F], Wxr[:F], Wxh[:F]], axis=1)
    WX1 = jnp.concatenate([Wxz[F:], Wxr[F:], Wxh[F:]], axis=1)
    WH0 = jnp.concatenate([Whz[:F], Whr[:F]], axis=1)
    WH1 = jnp.concatenate([Whz[F:], Whr[F:]], axis=1)
    bcat = jnp.concatenate([bxz + bhz, bxr + bhr, bxh + bhh]).reshape(1, 3 * F)

    degp = _sc_degree(dst_b, ones_blk, zer_deg)
    xw0, hw0 = _tc1a(X, H, WX0, WH0, bcat)
    xn, hn = _tc1b(X, H, degp)
    px = _sc_segsum1(xn, src_b, dst_b)
    ph = _sc_segsum1(hn, src_b, dst_b)
    z, hrn, t1 = _tc2(px, ph, degp, xw0, hw0, H, WX1, WH1, Whh[:F])
    phr = _sc_segsum1(hrn, src_b, dst_b)
    return _tc3(phr, degp, t1, z, H, Whh[F:])
